# baseline (speedup 1.0000x reference)
import functools

import jax
import jax.numpy as jnp
from jax import lax
from jax.experimental import pallas as pl
from jax.experimental.pallas import tpu as pltpu
from jax.experimental.pallas import tpu_sc as plsc

D = 32
L = 16
NC = 2
NS = 16
NW = NC * NS
K = 8
CH = 64
BCH = 256


def _dot_body(b_per_w, sid_hbm, pid_hbm, semb_hbm, pemb_hbm, out_hbm,
              idx_v, srows_v, prows_v, out_v, *sems):
    n_big = b_per_w // BCH
    wid = lax.axis_index("s") * NC + lax.axis_index("c")
    base = pl.multiple_of(wid * b_per_w, b_per_w)

    pltpu.sync_copy(sid_hbm.at[pl.ds(base, b_per_w)], idx_v.at[0])
    pltpu.sync_copy(pid_hbm.at[pl.ds(base, b_per_w)], idx_v.at[1])

    lane = lax.iota(jnp.int32, L)
    last_lane = lane == (L - 1)

    def big_chunk(c, carry):
        b0 = pl.multiple_of(c * BCH, BCH)

        def fire(f, carry2):
            r0 = pl.multiple_of(f * CH, CH)
            for g in range(CH // L):
                svec = idx_v[0, pl.ds(b0 + r0 + g * L, L)]
                pvec = idx_v[1, pl.ds(b0 + r0 + g * L, L)]
                for u in range(L):
                    j = g * L + u
                    pltpu.async_copy(
                        semb_hbm.at[pl.ds(svec[u], 1)],
                        srows_v.at[pl.ds(r0 + j, 1)], sems[j % K])
                    pltpu.async_copy(
                        pemb_hbm.at[pl.ds(pvec[u], 1)],
                        prows_v.at[pl.ds(r0 + j, 1)], sems[(j + K // 2) % K])
            return carry2

        lax.fori_loop(0, BCH // CH, fire, 0)

        for k in range(K):
            pltpu.make_async_copy(semb_hbm.at[pl.ds(0, BCH // K)],
                                  srows_v.at[pl.ds(0, BCH // K)],
                                  sems[k]).wait()
            pltpu.make_async_copy(pemb_hbm.at[pl.ds(0, BCH // K)],
                                  prows_v.at[pl.ds(0, BCH // K)],
                                  sems[k]).wait()

        def compute(g, carry2):
            r0 = pl.multiple_of(g * L, L)
            for u in range(L):
                q = (srows_v[r0 + u, pl.ds(0, L)]
                     * prows_v[r0 + u, pl.ds(0, L)]
                     + srows_v[r0 + u, pl.ds(L, L)]
                     * prows_v[r0 + u, pl.ds(L, L)])
                cum = plsc.cumsum(q)
                plsc.store_scatter(
                    out_v, [jnp.full((L,), b0 + r0 + u, jnp.int32)],
                    cum, mask=last_lane)
            return carry2

        lax.fori_loop(0, BCH // L, compute, 0)
        return carry

    lax.fori_loop(0, n_big, big_chunk, 0)
    pltpu.sync_copy(out_v, out_hbm.at[pl.ds(base, b_per_w)])


def kernel(sid, pid, scientist_emb, paper_emb):
    batch = sid.shape[0]
    b_per_w = batch // NW
    mesh = plsc.VectorSubcoreMesh(core_axis_name="c", subcore_axis_name="s",
                                  num_cores=NC, num_subcores=NS)
    k = pl.kernel(
        functools.partial(_dot_body, b_per_w),
        out_type=jax.ShapeDtypeStruct((batch,), jnp.float32),
        mesh=mesh,
        scratch_types=[
            pltpu.VMEM((2, b_per_w), jnp.int32),
            pltpu.VMEM((BCH, D), jnp.float32),
            pltpu.VMEM((BCH, D), jnp.float32),
            pltpu.VMEM((b_per_w,), jnp.float32),
        ] + [pltpu.SemaphoreType.DMA] * K,
        compiler_params=pltpu.CompilerParams(needs_layout_passes=False,
                                             use_tc_tiling_on_sc=True),
    )
    return k(sid.astype(jnp.int32), pid.astype(jnp.int32),
             scientist_emb, paper_emb)
